# baseline (speedup 1.0000x reference)
import functools

import jax
import jax.numpy as jnp
from jax import lax
from jax.experimental import pallas as pl
from jax.experimental.pallas import tpu as pltpu
from jax.experimental.pallas import tpu_sc as plsc

B, C, N, DK, DV, P, H = 8, 8, 512, 256, 256, 1024, 2
HD = DK // H
TJ = N
ROWS = B * C * N
NW = 32
RPW = ROWS // NW
CHUNK = 128
NCH = RPW // CHUNK

_INV_SQRT_HD = 1.0 / (HD ** 0.5)
_INV_SQRT2 = 0.7071067811865476


def _dot(a, b):
    return jnp.dot(a, b, preferred_element_type=jnp.float32)


def _rb(a):
    return a.astype(jnp.bfloat16).astype(jnp.float32)


def _layernorm(xm, w, b):
    m = jnp.mean(xm, axis=-1, keepdims=True)
    xc = xm - m
    v = jnp.mean(xc * xc, axis=-1, keepdims=True)
    return xc / jnp.sqrt(v + 1e-5) * w + b


def _gelu_exact(u):
    return u * 0.5 * (1.0 + lax.erf(u * _INV_SQRT2))


def _enc_body(x_ref, wiT, bi, woT, bo, l1w, l1b, l2w, l2b,
              w1T, b1, w2T, b2, wdT, bd, kT, kn, idx_ref):
    xb = x_ref[0]
    xmat = xb.reshape(C * TJ, DK)
    h = _layernorm(xmat, l1w[...], l1b[...])
    qkv = _dot(h, wiT[...]) + bi[...]

    q = [_rb(qkv[c * TJ:(c + 1) * TJ, 0:DK]) for c in range(C)]
    k = [_rb(qkv[c * TJ:(c + 1) * TJ, DK:2 * DK]) for c in range(C)]
    v = [_rb(qkv[c * TJ:(c + 1) * TJ, 2 * DK:3 * DK]) for c in range(C)]

    row = lax.broadcasted_iota(jnp.int32, (DK, H), 0)
    col = lax.broadcasted_iota(jnp.int32, (DK, H), 1)
    hsel = jnp.where(row // HD == col, 1.0, 0.0).astype(jnp.float32)

    av = []
    for c in range(C):
        pstack = jnp.concatenate([q[c] * k[e] for e in range(C)], axis=0)
        s2 = jnp.dot(pstack, hsel, preferred_element_type=jnp.float32,
                     precision=lax.Precision.HIGHEST) * _INV_SQRT_HD
        se = [s2[e * TJ:(e + 1) * TJ] for e in range(C)]
        mx = se[0]
        for e in range(1, C):
            mx = jnp.maximum(mx, se[e])
        ex = [jnp.exp(s - mx) for s in se]
        z = ex[0]
        for e in range(1, C):
            z = z + ex[e]
        at = [_rb(e_ / z) for e_ in ex]
        acc0 = at[0][:, 0:1] * v[0][:, :HD]
        acc1 = at[0][:, 1:2] * v[0][:, HD:]
        for e in range(1, C):
            acc0 = acc0 + at[e][:, 0:1] * v[e][:, :HD]
            acc1 = acc1 + at[e][:, 1:2] * v[e][:, HD:]
        av.append(jnp.concatenate([acc0, acc1], axis=1))
    avm = jnp.concatenate(av, axis=0)

    h2 = _dot(avm, woT[...]) + bo[...] + xmat
    g = _layernorm(h2, l2w[...], l2b[...])
    u = _gelu_exact(_dot(g, w1T[...]) + b1[...])
    f = _dot(u, w2T[...]) + b2[...] + h2
    fl = _dot(f, wdT[...]) + bd[...]

    for c in range(C):
        cross = _dot(fl[c * TJ:(c + 1) * TJ, :], kT[c])
        logits = 2.0 * cross - kn[c]
        best = jnp.argmax(logits, axis=-1).astype(jnp.int32)
        idx_ref[0, c, :] = best + c * P


@functools.partial(jax.jit, static_argnums=())
def _encode(x, wiT, bi, woT, bo, l1w, l1b, l2w, l2b,
            w1T, b1, w2T, b2, wdT, bd, kT, kn):
    full = lambda shape: pl.BlockSpec(shape, lambda b: (0,) * len(shape))
    return pl.pallas_call(
        _enc_body,
        grid=(B,),
        in_specs=[
            pl.BlockSpec((1, C, TJ, DK), lambda b: (b, 0, 0, 0)),
            full((DK, 3 * DK)), full((1, 3 * DK)),
            full((DK, DK)), full((1, DK)),
            full((1, DK)), full((1, DK)), full((1, DK)), full((1, DK)),
            full((DK, DK)), full((1, DK)),
            full((DK, DK)), full((1, DK)),
            full((DK, DK)), full((1, DK)),
            full((C, DK, P)), full((C, 1, P)),
        ],
        out_specs=pl.BlockSpec((1, C, N), lambda b: (b, 0, 0)),
        out_shape=jax.ShapeDtypeStruct((B, C, N), jnp.int32),
    )(x, wiT, bi, woT, bo, l1w, l1b, l2w, l2b,
      w1T, b1, w2T, b2, wdT, bd, kT, kn)


def _gather_values(idx_flat, values_flat):
    mesh = plsc.VectorSubcoreMesh(core_axis_name="c", subcore_axis_name="s")

    @functools.partial(
        pl.kernel, mesh=mesh,
        out_type=jax.ShapeDtypeStruct((ROWS, DV), jnp.float32),
        scratch_types=(
            [pltpu.VMEM((CHUNK,), jnp.int32) for _ in range(NCH)]
            + [pltpu.VMEM((CHUNK, DV), jnp.float32) for _ in range(3)]
            + [pltpu.SemaphoreType.DMA for _ in range(7)]
        ),
    )
    def gk(idx_hbm, val_hbm, out_hbm, *refs):
        idxs = refs[0:NCH]
        bufs = refs[NCH:NCH + 3]
        semi = refs[NCH + 3]
        gsems = refs[NCH + 4:NCH + 7]
        wsems = refs[NCH + 7:NCH + 10]
        wid = lax.axis_index("s") * 2 + lax.axis_index("c")
        base = wid * RPW
        ic = [pltpu.async_copy(idx_hbm.at[pl.ds(base + t * CHUNK, CHUNK)],
                               idxs[t], semi) for t in range(NCH)]
        for t in range(NCH):
            ic[t].wait()

        def gather(t):
            return pltpu.async_copy(val_hbm.at[idxs[t]], bufs[t % 3],
                                    gsems[t % 3])

        g = [None] * NCH
        w = [None] * NCH
        g[0] = gather(0)
        g[1] = gather(1)
        for i in range(NCH):
            if i == 0:
                g[2] = gather(2)
            elif i + 2 < NCH:
                w[i - 1].wait()
                g[i + 2] = gather(i + 2)
            g[i].wait()
            w[i] = pltpu.async_copy(
                bufs[i % 3], out_hbm.at[pl.ds(base + i * CHUNK, CHUNK)],
                wsems[i % 3])
        for t in range(max(0, NCH - 3), NCH):
            w[t].wait()

    return gk(idx_flat, values_flat)


def kernel(x, keys, values, in_proj_w, in_proj_b, out_w, out_b,
           ln1_w, ln1_b, ln2_w, ln2_b, W1, b1, W2, b2, Wd, bd):
    wiT = in_proj_w.T
    woT = out_w.T
    w1T = W1.T
    w2T = W2.T
    wdT = Wd.T
    kT = keys.transpose(0, 2, 1)
    kn = jnp.sum(keys * keys, axis=-1)[:, None, :]
    r2 = lambda a: a.reshape(1, -1)
    idx = _encode(x, wiT, r2(in_proj_b), woT, r2(out_b),
                  r2(ln1_w), r2(ln1_b), r2(ln2_w), r2(ln2_b),
                  w1T, r2(b1), w2T, r2(b2), wdT, r2(bd), kT, kn)
    out = _gather_values(idx.reshape(-1), values.reshape(C * P, DV))
    return out.reshape(B, C, N, DV)


# speedup vs baseline: 1.5163x; 1.5163x over previous
import functools

import jax
import jax.numpy as jnp
from jax import lax
from jax.experimental import pallas as pl
from jax.experimental.pallas import tpu as pltpu
from jax.experimental.pallas import tpu_sc as plsc

B, C, N, DK, DV, P, H = 8, 8, 512, 256, 256, 1024, 2
HD = DK // H
TJ = N
ROWS = B * C * N
NW = 32
RPW = ROWS // NW
CHUNK = 128
NCH = RPW // CHUNK

_INV_SQRT_HD = 1.0 / (HD ** 0.5)
_INV_SQRT2 = 0.7071067811865476


def _dot(a, b):
    return jnp.dot(a, b, preferred_element_type=jnp.float32)


def _rb(a):
    return a.astype(jnp.bfloat16).astype(jnp.float32)


def _layernorm(xm, w, b):
    m = jnp.mean(xm, axis=-1, keepdims=True)
    xc = xm - m
    v = jnp.mean(xc * xc, axis=-1, keepdims=True)
    return xc / jnp.sqrt(v + 1e-5) * w + b


def _gelu_exact(u):
    return u * 0.5 * (1.0 + lax.erf(u * _INV_SQRT2))


def _enc_body(x_ref, wiT, bi, woT, bo, l1w, l1b, l2w, l2b,
              w1T, b1, w2T, b2, wdT, bd, kT, kn, idx_ref):
    xb = x_ref[0]
    xmat = xb.reshape(C * TJ, DK)
    h = _layernorm(xmat, l1w[...], l1b[...])
    qkv = _dot(h, wiT[...]) + bi[...]

    q = [_rb(qkv[c * TJ:(c + 1) * TJ, 0:DK]) for c in range(C)]
    k = [_rb(qkv[c * TJ:(c + 1) * TJ, DK:2 * DK]) for c in range(C)]
    v = [_rb(qkv[c * TJ:(c + 1) * TJ, 2 * DK:3 * DK]) for c in range(C)]

    av = []
    for c in range(C):
        s0, s1 = [], []
        for e in range(C):
            prod = q[c] * k[e]
            s0.append(jnp.sum(prod[:, :HD], axis=1, keepdims=True))
            s1.append(jnp.sum(prod[:, HD:], axis=1, keepdims=True))
        a0 = jax.nn.softmax(jnp.concatenate(s0, axis=1) * _INV_SQRT_HD, axis=-1)
        a1 = jax.nn.softmax(jnp.concatenate(s1, axis=1) * _INV_SQRT_HD, axis=-1)
        a0 = _rb(a0)
        a1 = _rb(a1)
        acc0 = a0[:, 0:1] * v[0][:, :HD]
        acc1 = a1[:, 0:1] * v[0][:, HD:]
        for e in range(1, C):
            acc0 = acc0 + a0[:, e:e + 1] * v[e][:, :HD]
            acc1 = acc1 + a1[:, e:e + 1] * v[e][:, HD:]
        av.append(jnp.concatenate([acc0, acc1], axis=1))
    avm = jnp.concatenate(av, axis=0)

    h2 = _dot(avm, woT[...]) + bo[...] + xmat
    g = _layernorm(h2, l2w[...], l2b[...])
    u = _gelu_exact(_dot(g, w1T[...]) + b1[...])
    f = _dot(u, w2T[...]) + b2[...] + h2
    fl = _dot(f, wdT[...]) + bd[...]

    for c in range(C):
        cross = _dot(fl[c * TJ:(c + 1) * TJ, :], kT[c])
        logits = 2.0 * cross - kn[c]
        best = jnp.argmax(logits, axis=-1).astype(jnp.int32)
        idx_ref[0, c, :] = best + c * P


@functools.partial(jax.jit, static_argnums=())
def _encode(x, wiT, bi, woT, bo, l1w, l1b, l2w, l2b,
            w1T, b1, w2T, b2, wdT, bd, kT, kn):
    full = lambda shape: pl.BlockSpec(shape, lambda b: (0,) * len(shape))
    return pl.pallas_call(
        _enc_body,
        grid=(B,),
        in_specs=[
            pl.BlockSpec((1, C, TJ, DK), lambda b: (b, 0, 0, 0)),
            full((DK, 3 * DK)), full((1, 3 * DK)),
            full((DK, DK)), full((1, DK)),
            full((1, DK)), full((1, DK)), full((1, DK)), full((1, DK)),
            full((DK, DK)), full((1, DK)),
            full((DK, DK)), full((1, DK)),
            full((DK, DK)), full((1, DK)),
            full((C, DK, P)), full((C, 1, P)),
        ],
        out_specs=pl.BlockSpec((1, C, N), lambda b: (b, 0, 0)),
        out_shape=jax.ShapeDtypeStruct((B, C, N), jnp.int32),
    )(x, wiT, bi, woT, bo, l1w, l1b, l2w, l2b,
      w1T, b1, w2T, b2, wdT, bd, kT, kn)


def _gather_values(idx_flat, values_flat):
    mesh = plsc.VectorSubcoreMesh(core_axis_name="c", subcore_axis_name="s")

    nch0, nch1 = 6, 10
    nmax = max(nch0, nch1)

    @functools.partial(
        pl.kernel, mesh=mesh,
        out_type=jax.ShapeDtypeStruct((ROWS, DV), jnp.float32),
        scratch_types=(
            [pltpu.VMEM((CHUNK,), jnp.int32) for _ in range(nmax)]
            + [pltpu.VMEM((CHUNK, DV), jnp.float32) for _ in range(3)]
            + [pltpu.SemaphoreType.DMA for _ in range(7)]
        ),
    )
    def gk(idx_hbm, val_hbm, out_hbm, *refs):
        idxs = refs[0:nmax]
        bufs = refs[nmax:nmax + 3]
        semi = refs[nmax + 3]
        gsems = refs[nmax + 4:nmax + 7]
        wsems = refs[nmax + 7:nmax + 10]
        cid = lax.axis_index("c")
        sid = lax.axis_index("s")

        def run(base, nch):
            ic = [pltpu.async_copy(idx_hbm.at[pl.ds(base + t * CHUNK, CHUNK)],
                                   idxs[t], semi) for t in range(nch)]
            for t in range(nch):
                ic[t].wait()

            def gather(t):
                return pltpu.async_copy(val_hbm.at[idxs[t]], bufs[t % 3],
                                        gsems[t % 3])

            g = [None] * nch
            w = [None] * nch
            g[0] = gather(0)
            g[1] = gather(1)
            for i in range(nch):
                if i == 0:
                    g[2] = gather(2)
                elif i + 2 < nch:
                    w[i - 1].wait()
                    g[i + 2] = gather(i + 2)
                g[i].wait()
                w[i] = pltpu.async_copy(
                    bufs[i % 3], out_hbm.at[pl.ds(base + i * CHUNK, CHUNK)],
                    wsems[i % 3])
            for t in range(max(0, nch - 3), nch):
                w[t].wait()

        @pl.when(cid == 0)
        def _():
            run(sid * (nch0 * CHUNK), nch0)

        @pl.when(cid == 1)
        def _():
            run(16 * nch0 * CHUNK + sid * (nch1 * CHUNK), nch1)

    return gk(idx_flat, values_flat)


def kernel(x, keys, values, in_proj_w, in_proj_b, out_w, out_b,
           ln1_w, ln1_b, ln2_w, ln2_b, W1, b1, W2, b2, Wd, bd):
    wiT = in_proj_w.T
    woT = out_w.T
    w1T = W1.T
    w2T = W2.T
    wdT = Wd.T
    kT = keys.transpose(0, 2, 1)
    kn = jnp.sum(keys * keys, axis=-1)[:, None, :]
    r2 = lambda a: a.reshape(1, -1)
    idx = _encode(x, wiT, r2(in_proj_b), woT, r2(out_b),
                  r2(ln1_w), r2(ln1_b), r2(ln2_w), r2(ln2_b),
                  w1T, r2(b1), w2T, r2(b2), wdT, r2(bd), kT, kn)
    out = _gather_values(idx.reshape(-1), values.reshape(C * P, DV))
    return out.reshape(B, C, N, DV)


# speedup vs baseline: 1.6443x; 1.0844x over previous
import functools

import jax
import jax.numpy as jnp
from jax import lax
from jax.experimental import pallas as pl
from jax.experimental.pallas import tpu as pltpu
from jax.experimental.pallas import tpu_sc as plsc

B, C, N, DK, DV, P, H = 8, 8, 512, 256, 256, 1024, 2
HD = DK // H
TJ = N
ROWS = B * C * N
NW = 32
RPW = ROWS // NW
CHUNK = 128
NCH = RPW // CHUNK

_INV_SQRT_HD = 1.0 / (HD ** 0.5)
_INV_SQRT2 = 0.7071067811865476


def _dot(a, b):
    return jnp.dot(a, b, preferred_element_type=jnp.float32)


def _rb(a):
    return a.astype(jnp.bfloat16).astype(jnp.float32)


def _layernorm(xm, w, b):
    m = jnp.mean(xm, axis=-1, keepdims=True)
    xc = xm - m
    v = jnp.mean(xc * xc, axis=-1, keepdims=True)
    return xc / jnp.sqrt(v + 1e-5) * w + b


def _gelu_exact(u):
    return u * 0.5 * (1.0 + lax.erf(u * _INV_SQRT2))


def _enc_body(x_ref, wiT, bi, woT, bo, l1w, l1b, l2w, l2b,
              w1T, b1, w2T, b2, wdT, bd, kT, kn, idx_ref):
    xb = x_ref[0]
    xmat = xb.reshape(C * TJ, DK)
    h = _layernorm(xmat, l1w[...], l1b[...])
    qkv = _dot(h, wiT[...]) + bi[...]

    q = [_rb(qkv[c * TJ:(c + 1) * TJ, 0:DK]) for c in range(C)]
    k = [_rb(qkv[c * TJ:(c + 1) * TJ, DK:2 * DK]) for c in range(C)]
    v = [_rb(qkv[c * TJ:(c + 1) * TJ, 2 * DK:3 * DK]) for c in range(C)]

    av = []
    for c in range(C):
        s0, s1 = [], []
        for e in range(C):
            prod = q[c] * k[e]
            s0.append(jnp.sum(prod[:, :HD], axis=1, keepdims=True))
            s1.append(jnp.sum(prod[:, HD:], axis=1, keepdims=True))
        a0 = jax.nn.softmax(jnp.concatenate(s0, axis=1) * _INV_SQRT_HD, axis=-1)
        a1 = jax.nn.softmax(jnp.concatenate(s1, axis=1) * _INV_SQRT_HD, axis=-1)
        a0 = _rb(a0)
        a1 = _rb(a1)
        acc0 = a0[:, 0:1] * v[0][:, :HD]
        acc1 = a1[:, 0:1] * v[0][:, HD:]
        for e in range(1, C):
            acc0 = acc0 + a0[:, e:e + 1] * v[e][:, :HD]
            acc1 = acc1 + a1[:, e:e + 1] * v[e][:, HD:]
        av.append(jnp.concatenate([acc0, acc1], axis=1))
    avm = jnp.concatenate(av, axis=0)

    h2 = _dot(avm, woT[...]) + bo[...] + xmat
    g = _layernorm(h2, l2w[...], l2b[...])
    u = _gelu_exact(_dot(g, w1T[...]) + b1[...])
    f = _dot(u, w2T[...]) + b2[...] + h2
    fl = _dot(f, wdT[...]) + bd[...]

    for c in range(C):
        cross = _dot(fl[c * TJ:(c + 1) * TJ, :], kT[c])
        logits = 2.0 * cross - kn[c]
        best = jnp.argmax(logits, axis=-1).astype(jnp.int32)
        idx_ref[0, c, :] = best + c * P


def _encode(x, wiT, bi, woT, bo, l1w, l1b, l2w, l2b,
            w1T, b1, w2T, b2, wdT, bd, kT, kn, off, nb):
    full = lambda shape: pl.BlockSpec(shape, lambda b: (0,) * len(shape))
    return pl.pallas_call(
        _enc_body,
        grid=(nb,),
        in_specs=[
            pl.BlockSpec((1, C, TJ, DK), lambda b: (b + off, 0, 0, 0)),
            full((DK, 3 * DK)), full((1, 3 * DK)),
            full((DK, DK)), full((1, DK)),
            full((1, DK)), full((1, DK)), full((1, DK)), full((1, DK)),
            full((DK, DK)), full((1, DK)),
            full((DK, DK)), full((1, DK)),
            full((DK, DK)), full((1, DK)),
            full((C, DK, P)), full((C, 1, P)),
        ],
        out_specs=pl.BlockSpec((1, C, N), lambda b: (b, 0, 0)),
        out_shape=jax.ShapeDtypeStruct((nb, C, N), jnp.int32),
    )(x, wiT, bi, woT, bo, l1w, l1b, l2w, l2b,
      w1T, b1, w2T, b2, wdT, bd, kT, kn)


def _gather_values(idx_flat, values_flat, nch0, nch1):
    mesh = plsc.VectorSubcoreMesh(core_axis_name="c", subcore_axis_name="s")

    nrows = (nch0 + nch1) * 16 * CHUNK
    nmax = max(nch0, nch1)

    @functools.partial(
        pl.kernel, mesh=mesh,
        out_type=jax.ShapeDtypeStruct((nrows, DV), jnp.float32),
        scratch_types=(
            [pltpu.VMEM((CHUNK,), jnp.int32) for _ in range(nmax)]
            + [pltpu.VMEM((CHUNK, DV), jnp.float32) for _ in range(3)]
            + [pltpu.SemaphoreType.DMA for _ in range(7)]
        ),
    )
    def gk(idx_hbm, val_hbm, out_hbm, *refs):
        idxs = refs[0:nmax]
        bufs = refs[nmax:nmax + 3]
        semi = refs[nmax + 3]
        gsems = refs[nmax + 4:nmax + 7]
        wsems = refs[nmax + 7:nmax + 10]
        cid = lax.axis_index("c")
        sid = lax.axis_index("s")

        def run(base, nch):
            ic = [pltpu.async_copy(idx_hbm.at[pl.ds(base + t * CHUNK, CHUNK)],
                                   idxs[t], semi) for t in range(nch)]
            for t in range(nch):
                ic[t].wait()

            def gather(t):
                return pltpu.async_copy(val_hbm.at[idxs[t]], bufs[t % 3],
                                        gsems[t % 3])

            g = [None] * nch
            w = [None] * nch
            g[0] = gather(0)
            g[1] = gather(1)
            for i in range(nch):
                if i == 0:
                    g[2] = gather(2)
                elif i + 2 < nch:
                    w[i - 1].wait()
                    g[i + 2] = gather(i + 2)
                g[i].wait()
                w[i] = pltpu.async_copy(
                    bufs[i % 3], out_hbm.at[pl.ds(base + i * CHUNK, CHUNK)],
                    wsems[i % 3])
            for t in range(max(0, nch - 3), nch):
                w[t].wait()

        @pl.when(cid == 0)
        def _():
            run(sid * (nch0 * CHUNK), nch0)

        @pl.when(cid == 1)
        def _():
            run(16 * nch0 * CHUNK + sid * (nch1 * CHUNK), nch1)

    return gk(idx_flat, values_flat)


def kernel(x, keys, values, in_proj_w, in_proj_b, out_w, out_b,
           ln1_w, ln1_b, ln2_w, ln2_b, W1, b1, W2, b2, Wd, bd):
    wiT = in_proj_w.T
    woT = out_w.T
    w1T = W1.T
    w2T = W2.T
    wdT = Wd.T
    kT = keys.transpose(0, 2, 1)
    kn = jnp.sum(keys * keys, axis=-1)[:, None, :]
    r2 = lambda a: a.reshape(1, -1)
    vflat = values.reshape(C * P, DV)
    weights = (wiT, r2(in_proj_b), woT, r2(out_b),
               r2(ln1_w), r2(ln1_b), r2(ln2_w), r2(ln2_b),
               w1T, r2(b1), w2T, r2(b2), wdT, r2(bd), kT, kn)
    hb = B // 2
    idx_a = _encode(x, *weights, 0, hb)
    out_a = _gather_values(idx_a.reshape(-1), vflat, 3, 5)
    idx_b = _encode(x, *weights, hb, hb)
    out_b = _gather_values(idx_b.reshape(-1), vflat, 3, 5)
    out = jnp.concatenate([out_a, out_b], axis=0)
    return out.reshape(B, C, N, DV)
